# baseline (speedup 1.0000x reference)
import functools

import jax
import jax.numpy as jnp
from jax import lax
from jax.experimental import pallas as pl
from jax.experimental.pallas import tpu as pltpu
from jax.experimental.pallas import tpu_sc as plsc

N_NODES = 10000
D = 128
E = 320000

NC = 2
NS = 16
NW = NC * NS
L = 16

CHUNK = 40
EDGES_PER_TILE = E // NW
NCHUNK = EDGES_PER_TILE // CHUNK
NPAD = 10112
ROWS_PER_TILE = NPAD // NS


def _sc_accumulate(src_hbm, dst_hbm, x_hbm, zf_hbm,
                   partial_hbm, cntw_hbm,
                   src_v, dst_v, rows_v, cnt_v, acc_sh, sem_a, sem_b):
    c = lax.axis_index("c")
    s = lax.axis_index("s")
    wid = s * NC + c
    row0 = s * ROWS_PER_TILE

    pltpu.sync_copy(zf_hbm, acc_sh.at[pl.ds(row0, ROWS_PER_TILE)])
    pltpu.sync_copy(src_hbm.at[wid], src_v)
    pltpu.sync_copy(dst_hbm.at[wid], dst_v)

    zv = jnp.zeros((L,), jnp.float32)

    def zbody(k, carry):
        cnt_v[pl.ds(k * L, L)] = zv
        return carry

    lax.fori_loop(0, NPAD // L, zbody, 0)

    ones_v = jnp.ones((L,), jnp.float32)

    def cbody(k, carry):
        dv = dst_v[pl.ds(k * L, L)]
        plsc.addupdate_scatter(cnt_v, [dv], ones_v)
        return carry

    lax.fori_loop(0, EDGES_PER_TILE // L, cbody, 0)
    plsc.subcore_barrier()

    def gather(j, slot, sem):
        return pltpu.async_copy(
            x_hbm.at[src_v.at[pl.ds(j * CHUNK, CHUNK)]], rows_v.at[slot], sem)

    gather(0, 0, sem_a)

    def body(j, carry):
        even = (j % 2) == 0

        def step(slot_cur, slot_nxt, sem_cur, sem_nxt):
            pltpu.make_async_copy(
                x_hbm.at[src_v.at[pl.ds(j * CHUNK, CHUNK)]],
                rows_v.at[slot_cur], sem_cur).wait()

            @pl.when(j + 1 < NCHUNK)
            def _():
                gather(j + 1, slot_nxt, sem_nxt)

            pltpu.sync_copy(rows_v.at[slot_cur],
                            acc_sh.at[dst_v.at[pl.ds(j * CHUNK, CHUNK)]],
                            add=True)

        @pl.when(even)
        def _():
            step(0, 1, sem_a, sem_b)

        @pl.when(jnp.logical_not(even))
        def _():
            step(1, 0, sem_b, sem_a)

        return carry

    lax.fori_loop(0, NCHUNK, body, 0)
    plsc.subcore_barrier()

    pltpu.sync_copy(acc_sh.at[pl.ds(row0, ROWS_PER_TILE)],
                    partial_hbm.at[c, pl.ds(row0, ROWS_PER_TILE)])
    pltpu.sync_copy(cnt_v, cntw_hbm.at[wid])


def _combine_body(p_ref, c_ref, o_ref):
    p = p_ref[0] + p_ref[1]
    cnt = jnp.sum(c_ref[...], axis=0)
    o_ref[...] = p * (1.0 / jnp.maximum(cnt, 1.0))[:, None]


def kernel(x, edge_index):
    x = x.astype(jnp.float32)
    ei = edge_index.astype(jnp.int32)
    src = ei[0].reshape(NW, EDGES_PER_TILE)
    dst = ei[1].reshape(NW, EDGES_PER_TILE)

    zf = jnp.zeros((ROWS_PER_TILE, D), jnp.float32)

    mesh = plsc.VectorSubcoreMesh(core_axis_name="c", subcore_axis_name="s")
    sc_fn = functools.partial(
        pl.kernel,
        mesh=mesh,
        compiler_params=pltpu.CompilerParams(needs_layout_passes=False),
        out_type=[
            jax.ShapeDtypeStruct((NC, NPAD, D), jnp.float32),
            jax.ShapeDtypeStruct((NW, NPAD), jnp.float32),
        ],
        scratch_types=[
            pltpu.VMEM((EDGES_PER_TILE,), jnp.int32),
            pltpu.VMEM((EDGES_PER_TILE,), jnp.int32),
            pltpu.VMEM((2, CHUNK, D), jnp.float32),
            pltpu.VMEM((NPAD,), jnp.float32),
            pltpu.VMEM_SHARED((NPAD, D), jnp.float32),
            pltpu.SemaphoreType.DMA,
            pltpu.SemaphoreType.DMA,
        ],
    )(_sc_accumulate)
    partial, cntw = sc_fn(src, dst, x, zf)

    h = pl.pallas_call(
        _combine_body,
        out_shape=jax.ShapeDtypeStruct((NPAD, D), jnp.float32),
    )(partial, cntw)
    return h[:N_NODES]
